# Initial kernel scaffold; baseline (speedup 1.0000x reference)
import jax
import jax.numpy as jnp
from jax.experimental import pallas as pl


def kernel(x, R, centroids):
    raise NotImplementedError("write your pallas kernel here")


# speedup vs baseline: 1.9022x; 1.9022x over previous
import jax
import jax.numpy as jnp
from jax.experimental import pallas as pl
from jax.experimental.pallas import tpu as pltpu

_D = 128
_NUM_STAGES = 4
_BLOCK = 2048


def _msrc_kernel(c_ref, x_ref, r_ref, o_ref):
    x = x_ref[...]
    R = r_ref[...]
    nrm = jnp.sqrt(jnp.sum(x * x, axis=1, keepdims=True))
    xn = x / (nrm + 1e-8)
    xr = jax.lax.dot_general(
        xn, R, (((1,), (0,)), ((), ())),
        preferred_element_type=jnp.float32)
    residual = xr
    combined = jnp.zeros_like(xr)
    for s in range(_NUM_STAGES):
        neg = c_ref[s, 0]
        pos = c_ref[s, 1]
        recon = jnp.where(residual > 0, pos, neg)
        combined = combined + recon
        residual = residual - recon
    out = jax.lax.dot_general(
        combined, R, (((1,), (1,)), ((), ())), preferred_element_type=jnp.float32)
    o_ref[...] = out * nrm


def kernel(x, R, centroids):
    n, d = x.shape
    grid = (n // _BLOCK,)
    return pl.pallas_call(
        _msrc_kernel,
        grid=grid,
        in_specs=[
            pl.BlockSpec(memory_space=pltpu.SMEM),
            pl.BlockSpec((_BLOCK, d), lambda i: (i, 0)),
            pl.BlockSpec((d, d), lambda i: (0, 0)),
        ],
        out_specs=pl.BlockSpec((_BLOCK, d), lambda i: (i, 0)),
        out_shape=jax.ShapeDtypeStruct((n, d), jnp.float32),
        compiler_params=pltpu.CompilerParams(
            dimension_semantics=("arbitrary",)),
    )(centroids, x, R)


# speedup vs baseline: 1.9243x; 1.0116x over previous
import jax
import jax.numpy as jnp
from jax.experimental import pallas as pl
from jax.experimental.pallas import tpu as pltpu

_D = 128
_NUM_STAGES = 4
_BLOCK = 2048


def _msrc_kernel(c_ref, x_ref, r_ref, o_ref):
    x = x_ref[...]
    R = r_ref[...]
    nrm = jnp.sqrt(jnp.sum(x * x, axis=1, keepdims=True))
    xn = x / (nrm + 1e-8)
    xr = jax.lax.dot_general(
        xn, R, (((1,), (0,)), ((), ())),
        preferred_element_type=jnp.float32)
    sign_mask = jnp.int32(-2147483648)
    residual = xr
    for s in range(_NUM_STAGES):
        pos_bits = jax.lax.bitcast_convert_type(c_ref[s, 1], jnp.int32)
        r_bits = jax.lax.bitcast_convert_type(residual, jnp.int32)
        recon = jax.lax.bitcast_convert_type(
            (r_bits & sign_mask) ^ pos_bits, jnp.float32)
        residual = residual - recon
    combined = xr - residual
    out = jax.lax.dot_general(
        combined, R, (((1,), (1,)), ((), ())), preferred_element_type=jnp.float32)
    o_ref[...] = out * nrm


def kernel(x, R, centroids):
    n, d = x.shape
    grid = (n // _BLOCK,)
    return pl.pallas_call(
        _msrc_kernel,
        grid=grid,
        in_specs=[
            pl.BlockSpec(memory_space=pltpu.SMEM),
            pl.BlockSpec((_BLOCK, d), lambda i: (i, 0)),
            pl.BlockSpec((d, d), lambda i: (0, 0)),
        ],
        out_specs=pl.BlockSpec((_BLOCK, d), lambda i: (i, 0)),
        out_shape=jax.ShapeDtypeStruct((n, d), jnp.float32),
        compiler_params=pltpu.CompilerParams(
            dimension_semantics=("arbitrary",)),
    )(centroids, x, R)


# speedup vs baseline: 2.0303x; 1.0551x over previous
import jax
import jax.numpy as jnp
from jax.experimental import pallas as pl
from jax.experimental.pallas import tpu as pltpu

_D = 128
_NUM_STAGES = 4
_BLOCK = 2048


def _msrc_kernel(c_ref, x_ref, r_ref, o_ref):
    x = x_ref[...]
    R = r_ref[...]
    ssq = jnp.sum(x * x, axis=1, keepdims=True)
    inv = jax.lax.rsqrt(ssq)
    nrm = ssq * inv
    xn = x * inv
    xr = jax.lax.dot_general(
        xn, R, (((1,), (0,)), ((), ())),
        preferred_element_type=jnp.float32)
    sign_mask = jnp.int32(-2147483648)
    residual = xr
    for s in range(_NUM_STAGES):
        pos_bits = jax.lax.bitcast_convert_type(c_ref[s, 1], jnp.int32)
        r_bits = jax.lax.bitcast_convert_type(residual, jnp.int32)
        recon = jax.lax.bitcast_convert_type(
            (r_bits & sign_mask) ^ pos_bits, jnp.float32)
        residual = residual - recon
    combined = xr - residual
    out = jax.lax.dot_general(
        combined, R, (((1,), (1,)), ((), ())), preferred_element_type=jnp.float32)
    o_ref[...] = out * nrm


def kernel(x, R, centroids):
    n, d = x.shape
    grid = (n // _BLOCK,)
    return pl.pallas_call(
        _msrc_kernel,
        grid=grid,
        in_specs=[
            pl.BlockSpec(memory_space=pltpu.SMEM),
            pl.BlockSpec((_BLOCK, d), lambda i: (i, 0)),
            pl.BlockSpec((d, d), lambda i: (0, 0)),
        ],
        out_specs=pl.BlockSpec((_BLOCK, d), lambda i: (i, 0)),
        out_shape=jax.ShapeDtypeStruct((n, d), jnp.float32),
        compiler_params=pltpu.CompilerParams(
            dimension_semantics=("arbitrary",)),
    )(centroids, x, R)


# speedup vs baseline: 2.5800x; 1.2708x over previous
import jax
import jax.numpy as jnp
from jax.experimental import pallas as pl
from jax.experimental.pallas import tpu as pltpu

_D = 128
_NUM_STAGES = 4
_BLOCK = 4096


def _msrc_kernel(c_ref, x_ref, r_ref, o_ref):
    x = x_ref[...]
    R = r_ref[...]
    ssq = jnp.sum(x * x, axis=1, keepdims=True)
    inv = jax.lax.rsqrt(ssq)
    nrm = ssq * inv
    xn = x * inv
    xr = jax.lax.dot_general(
        xn, R, (((1,), (0,)), ((), ())),
        preferred_element_type=jnp.float32)
    sign_mask = jnp.int32(-2147483648)
    residual = xr
    for s in range(_NUM_STAGES):
        pos_bits = jax.lax.bitcast_convert_type(c_ref[s, 1], jnp.int32)
        r_bits = jax.lax.bitcast_convert_type(residual, jnp.int32)
        recon = jax.lax.bitcast_convert_type(
            (r_bits & sign_mask) ^ pos_bits, jnp.float32)
        residual = residual - recon
    combined = xr - residual
    out = jax.lax.dot_general(
        combined, R, (((1,), (1,)), ((), ())), preferred_element_type=jnp.float32)
    o_ref[...] = out * nrm


def kernel(x, R, centroids):
    n, d = x.shape
    grid = (n // _BLOCK,)
    return pl.pallas_call(
        _msrc_kernel,
        grid=grid,
        in_specs=[
            pl.BlockSpec(memory_space=pltpu.SMEM),
            pl.BlockSpec((_BLOCK, d), lambda i: (i, 0)),
            pl.BlockSpec((d, d), lambda i: (0, 0)),
        ],
        out_specs=pl.BlockSpec((_BLOCK, d), lambda i: (i, 0)),
        out_shape=jax.ShapeDtypeStruct((n, d), jnp.float32),
        compiler_params=pltpu.CompilerParams(
            dimension_semantics=("arbitrary",)),
    )(centroids, x, R)


# speedup vs baseline: 2.9628x; 1.1483x over previous
import jax
import jax.numpy as jnp
from jax.experimental import pallas as pl
from jax.experimental.pallas import tpu as pltpu

_D = 128
_NUM_STAGES = 4
_BLOCK = 8192


def _msrc_kernel(c_ref, x_ref, r_ref, o_ref):
    x = x_ref[...]
    R = r_ref[...]
    ssq = jnp.sum(x * x, axis=1, keepdims=True)
    inv = jax.lax.rsqrt(ssq)
    nrm = ssq * inv
    xn = x * inv
    xr = jax.lax.dot_general(
        xn, R, (((1,), (0,)), ((), ())),
        preferred_element_type=jnp.float32)
    sign_mask = jnp.int32(-2147483648)
    residual = xr
    for s in range(_NUM_STAGES):
        pos_bits = jax.lax.bitcast_convert_type(c_ref[s, 1], jnp.int32)
        r_bits = jax.lax.bitcast_convert_type(residual, jnp.int32)
        recon = jax.lax.bitcast_convert_type(
            (r_bits & sign_mask) ^ pos_bits, jnp.float32)
        residual = residual - recon
    combined = xr - residual
    out = jax.lax.dot_general(
        combined, R, (((1,), (1,)), ((), ())), preferred_element_type=jnp.float32)
    o_ref[...] = out * nrm


def kernel(x, R, centroids):
    n, d = x.shape
    grid = (n // _BLOCK,)
    return pl.pallas_call(
        _msrc_kernel,
        grid=grid,
        in_specs=[
            pl.BlockSpec(memory_space=pltpu.SMEM),
            pl.BlockSpec((_BLOCK, d), lambda i: (i, 0)),
            pl.BlockSpec((d, d), lambda i: (0, 0)),
        ],
        out_specs=pl.BlockSpec((_BLOCK, d), lambda i: (i, 0)),
        out_shape=jax.ShapeDtypeStruct((n, d), jnp.float32),
        compiler_params=pltpu.CompilerParams(
            dimension_semantics=("arbitrary",)),
    )(centroids, x, R)


# speedup vs baseline: 3.0233x; 1.0204x over previous
import jax
import jax.numpy as jnp
from jax.experimental import pallas as pl
from jax.experimental.pallas import tpu as pltpu

_D = 128
_NUM_STAGES = 4
_BLOCK = 16384


def _msrc_kernel(c_ref, x_ref, r_ref, o_ref):
    x = x_ref[...]
    R = r_ref[...]
    ssq = jnp.sum(x * x, axis=1, keepdims=True)
    inv = jax.lax.rsqrt(ssq)
    nrm = ssq * inv
    xn = x * inv
    xr = jax.lax.dot_general(
        xn, R, (((1,), (0,)), ((), ())),
        preferred_element_type=jnp.float32)
    sign_mask = jnp.int32(-2147483648)
    residual = xr
    for s in range(_NUM_STAGES):
        pos_bits = jax.lax.bitcast_convert_type(c_ref[s, 1], jnp.int32)
        r_bits = jax.lax.bitcast_convert_type(residual, jnp.int32)
        recon = jax.lax.bitcast_convert_type(
            (r_bits & sign_mask) ^ pos_bits, jnp.float32)
        residual = residual - recon
    combined = xr - residual
    out = jax.lax.dot_general(
        combined, R, (((1,), (1,)), ((), ())), preferred_element_type=jnp.float32)
    o_ref[...] = out * nrm


def kernel(x, R, centroids):
    n, d = x.shape
    grid = (n // _BLOCK,)
    return pl.pallas_call(
        _msrc_kernel,
        grid=grid,
        in_specs=[
            pl.BlockSpec(memory_space=pltpu.SMEM),
            pl.BlockSpec((_BLOCK, d), lambda i: (i, 0)),
            pl.BlockSpec((d, d), lambda i: (0, 0)),
        ],
        out_specs=pl.BlockSpec((_BLOCK, d), lambda i: (i, 0)),
        out_shape=jax.ShapeDtypeStruct((n, d), jnp.float32),
        compiler_params=pltpu.CompilerParams(
            dimension_semantics=("arbitrary",)),
    )(centroids, x, R)
